# Initial kernel scaffold; baseline (speedup 1.0000x reference)
import jax
import jax.numpy as jnp
from jax.experimental import pallas as pl


def kernel(x, edge_index, W1, b1, W2, b2):
    raise NotImplementedError("write your pallas kernel here")


# speedup vs baseline: 1771.8914x; 1771.8914x over previous
import functools

import jax
import jax.numpy as jnp
from jax import lax
from jax.experimental import pallas as pl
from jax.experimental.pallas import tpu as pltpu
from jax.experimental.pallas import tpu_sc as plsc

NC = 2
NS = 16
CHUNK = 128


def _mesh():
    return plsc.VectorSubcoreMesh(core_axis_name="c", subcore_axis_name="s")


def _edge_gather(x, col0, col1):
    N, D = x.shape
    E = col0.shape[0]
    per = E // (NC * NS)
    nch = per // CHUNK

    @functools.partial(
        pl.kernel,
        out_type=(jax.ShapeDtypeStruct((E, D), jnp.float32),
                  jax.ShapeDtypeStruct((E, D), jnp.float32)),
        mesh=_mesh(),
        scratch_types=[pltpu.VMEM((CHUNK,), jnp.int32),
                       pltpu.VMEM((CHUNK, D), jnp.float32),
                       pltpu.SemaphoreType.DMA],
    )
    def k(x_hbm, c0_hbm, c1_hbm, xs_hbm, xd_hbm, idx_v, buf, sem):
        cid = lax.axis_index("c")
        sid = lax.axis_index("s")
        wid = cid * NS + sid
        for ci in range(nch):
            base = wid * per + ci * CHUNK
            for src_idx, dst in ((c0_hbm, xs_hbm), (c1_hbm, xd_hbm)):
                pltpu.sync_copy(src_idx.at[pl.ds(base, CHUNK)], idx_v)
                pltpu.async_copy(x_hbm.at[idx_v], buf, sem).wait()
                pltpu.sync_copy(buf, dst.at[pl.ds(base, CHUNK)])

    return k(x, col0, col1)


def _sc_round(msg, sidx, col0, zblk, npad):
    E, D = msg.shape
    per = E // NS
    nch = per // CHUNK
    rpt = npad // NS

    @functools.partial(
        pl.kernel,
        out_type=jax.ShapeDtypeStruct((E, D), jnp.float32),
        mesh=_mesh(),
        scratch_types=[pltpu.VMEM_SHARED((npad, D), jnp.float32),
                       pltpu.VMEM((CHUNK,), jnp.int32),
                       pltpu.VMEM((CHUNK, D), jnp.float32),
                       pltpu.SemaphoreType.DMA],
    )
    def k(msg_hbm, sidx_hbm, c0_hbm, z_hbm, aggr_hbm, S, idx_v, buf, sem):
        cid = lax.axis_index("c")
        sid = lax.axis_index("s")

        @pl.when(cid == 0)
        def _():
            pltpu.sync_copy(z_hbm, buf)
            r0 = sid * rpt
            for off in range(0, rpt, CHUNK):
                pltpu.sync_copy(buf, S.at[pl.ds(r0 + off, CHUNK)])
            plsc.subcore_barrier()
            for ci in range(nch):
                base = sid * per + ci * CHUNK
                pltpu.sync_copy(sidx_hbm.at[pl.ds(base, CHUNK)], idx_v)
                pltpu.sync_copy(msg_hbm.at[pl.ds(base, CHUNK)], buf)
                pltpu.sync_copy(buf, S.at[idx_v], add=True)
            plsc.subcore_barrier()
            for ci in range(nch):
                base = sid * per + ci * CHUNK
                pltpu.sync_copy(c0_hbm.at[pl.ds(base, CHUNK)], idx_v)
                pltpu.async_copy(S.at[idx_v], buf, sem).wait()
                pltpu.sync_copy(buf, aggr_hbm.at[pl.ds(base, CHUNK)])

    return k(msg, sidx, col0, zblk)


def _sc_final(lgX, col1, zblk, oblk, npad):
    E, D = lgX.shape
    per = E // NS
    nch = per // CHUNK
    rpt = npad // NS

    @functools.partial(
        pl.kernel,
        out_type=(jax.ShapeDtypeStruct((npad, D), jnp.float32),
                  jax.ShapeDtypeStruct((npad, D), jnp.float32)),
        mesh=_mesh(),
        scratch_types=[pltpu.VMEM_SHARED((npad, D), jnp.float32),
                       pltpu.VMEM((CHUNK,), jnp.int32),
                       pltpu.VMEM((CHUNK, D), jnp.float32),
                       pltpu.SemaphoreType.DMA],
    )
    def k(lg_hbm, c1_hbm, z_hbm, o_hbm, sums_hbm, cnts_hbm, S, idx_v, buf, sem):
        cid = lax.axis_index("c")
        sid = lax.axis_index("s")
        pltpu.sync_copy(z_hbm, buf)
        r0 = sid * rpt
        for off in range(0, rpt, CHUNK):
            pltpu.sync_copy(buf, S.at[pl.ds(r0 + off, CHUNK)])
        plsc.subcore_barrier()

        @pl.when(cid == 1)
        def _():
            pltpu.sync_copy(o_hbm, buf)

        for ci in range(nch):
            base = sid * per + ci * CHUNK
            pltpu.sync_copy(c1_hbm.at[pl.ds(base, CHUNK)], idx_v)

            @pl.when(cid == 0)
            def _():
                pltpu.sync_copy(lg_hbm.at[pl.ds(base, CHUNK)], buf)

            pltpu.sync_copy(buf, S.at[idx_v], add=True)
        plsc.subcore_barrier()
        for off in range(0, rpt, CHUNK):
            pltpu.sync_copy(S.at[pl.ds(r0 + off, CHUNK)], buf)

            @pl.when(cid == 0)
            def _():
                pltpu.sync_copy(buf, sums_hbm.at[pl.ds(r0 + off, CHUNK)])

            @pl.when(cid == 1)
            def _():
                pltpu.sync_copy(buf, cnts_hbm.at[pl.ds(r0 + off, CHUNK)])

    return k(lgX, col1, zblk, oblk)


_EBLK = 1024


def _tc_prep(xs, xd):
    E, D = xs.shape

    def body(xs_ref, xd_ref, lg_ref, msg_ref):
        s = xs_ref[...]
        d = xd_ref[...]
        lg = (s + d) * 0.5
        lg_ref[...] = lg
        msg_ref[...] = jnp.maximum(lg + d, 0.0)

    return pl.pallas_call(
        body,
        grid=(E // _EBLK,),
        in_specs=[pl.BlockSpec((_EBLK, D), lambda i: (i, 0))] * 2,
        out_specs=[pl.BlockSpec((_EBLK, D), lambda i: (i, 0))] * 2,
        out_shape=[jax.ShapeDtypeStruct((E, D), jnp.float32)] * 2,
    )(xs, xd)


def _tc_update(lgX, aggr, ea, W1, b1r, W2, b2r):
    E, D = lgX.shape

    def body(lg_ref, ag_ref, ea_ref, w1_ref, b1_ref, w2_ref, b2_ref,
             out_ref, msg_ref):
        h = lg_ref[...] + ag_ref[...]
        t = jnp.dot(h, w1_ref[...], preferred_element_type=jnp.float32)
        t = jnp.maximum(t + b1_ref[...], 0.0)
        o = jnp.dot(t, w2_ref[...], preferred_element_type=jnp.float32)
        o = o + b2_ref[...]
        out_ref[...] = o
        msg_ref[...] = jnp.maximum(o + ea_ref[...], 0.0)

    eb = pl.BlockSpec((_EBLK, D), lambda i: (i, 0))
    return pl.pallas_call(
        body,
        grid=(E // _EBLK,),
        in_specs=[eb, eb, eb,
                  pl.BlockSpec((D, D), lambda i: (0, 0)),
                  pl.BlockSpec((1, D), lambda i: (0, 0)),
                  pl.BlockSpec((D, D), lambda i: (0, 0)),
                  pl.BlockSpec((1, D), lambda i: (0, 0))],
        out_specs=[eb, eb],
        out_shape=[jax.ShapeDtypeStruct((E, D), jnp.float32)] * 2,
    )(lgX, aggr, ea, W1, b1r, W2, b2r)


def _tc_final(sums, cnts):
    npad, D = sums.shape

    def body(s_ref, c_ref, o_ref):
        c = jnp.maximum(c_ref[...], 1.0)
        o_ref[...] = jnp.maximum(s_ref[...] / c, 0.0)

    nb = pl.BlockSpec((_EBLK, D), lambda i: (i, 0))
    return pl.pallas_call(
        body,
        grid=(npad // _EBLK,),
        in_specs=[nb, nb],
        out_specs=nb,
        out_shape=jax.ShapeDtypeStruct((npad, D), jnp.float32),
    )(sums, cnts)


def kernel(x, edge_index, W1, b1, W2, b2):
    N, D = x.shape
    E = edge_index.shape[1]
    col0 = edge_index[0]
    col1 = edge_index[1]
    sidx = jnp.where(col0 != col1, col1, jnp.int32(N))
    npad = -(-(N + 1) // (NS * CHUNK)) * (NS * CHUNK)
    zblk = jnp.zeros((CHUNK, D), jnp.float32)
    oblk = jnp.ones((CHUNK, D), jnp.float32)
    b1r = b1.reshape(1, D)
    b2r = b2.reshape(1, D)

    xs, xd = _edge_gather(x, col0, col1)
    lgX, msg = _tc_prep(xs, xd)
    for _ in range(3):
        aggr = _sc_round(msg, sidx, col0, zblk, npad)
        lgX, msg = _tc_update(lgX, aggr, xd, W1, b1r, W2, b2r)
    sums, cnts = _sc_final(lgX, col1, zblk, oblk, npad)
    out = _tc_final(sums, cnts)
    return out[:N]


# speedup vs baseline: 2268.9685x; 1.2805x over previous
import functools

import jax
import jax.numpy as jnp
from jax import lax
from jax.experimental import pallas as pl
from jax.experimental.pallas import tpu as pltpu
from jax.experimental.pallas import tpu_sc as plsc

NC = 2
NS = 16
CHUNK = 128
NB = 3
NBS = 2


def _mesh():
    return plsc.VectorSubcoreMesh(core_axis_name="c", subcore_axis_name="s")


def _edge_gather(x, c0r, c1r):
    N, D = x.shape
    E = c0r.shape[0] * CHUNK
    per = E // (NC * NS)
    nch = per // CHUNK
    iw = 2 * nch

    @functools.partial(
        pl.kernel,
        out_type=(jax.ShapeDtypeStruct((E, D), jnp.float32),
                  jax.ShapeDtypeStruct((E, D), jnp.float32)),
        mesh=_mesh(),
        scratch_types=[pltpu.VMEM((iw, CHUNK), jnp.int32),
                       pltpu.VMEM((NB, CHUNK, D), jnp.float32),
                       pltpu.SemaphoreType.DMA,
                       pltpu.SemaphoreType.DMA,
                       pltpu.SemaphoreType.DMA],
    )
    def k(x_hbm, c0_hbm, c1_hbm, xs_hbm, xd_hbm, idx_v, rbuf, si, sg, sw):
        cid = lax.axis_index("c")
        sid = lax.axis_index("s")
        wid = cid * NS + sid
        row0 = wid * nch
        d0 = pltpu.async_copy(c0_hbm.at[pl.ds(row0, nch)],
                              idx_v.at[pl.ds(0, nch)], si)
        d1 = pltpu.async_copy(c1_hbm.at[pl.ds(row0, nch)],
                              idx_v.at[pl.ds(nch, nch)], si)
        d0.wait()
        d1.wait()
        wdescs = []
        for c in range(iw):
            s = c % NB
            if c >= NB:
                wdescs[c - NB].wait()
            dst = xs_hbm if c < nch else xd_hbm
            base = wid * per + (c % nch) * CHUNK
            pltpu.async_copy(x_hbm.at[idx_v.at[c]], rbuf.at[s], sg).wait()
            wdescs.append(
                pltpu.async_copy(rbuf.at[s], dst.at[pl.ds(base, CHUNK)], sw))
        for d in wdescs[max(0, iw - NB):]:
            d.wait()

    return k(x, c0r, c1r)


def _sc_round(msg, sidxr, c0r, zeros_hbm, npad):
    E, D = msg.shape
    per_s = E // NS
    nch_s = per_s // CHUNK
    per_g = E // (NC * NS)
    nch_g = per_g // CHUNK
    rpt = npad // NS

    @functools.partial(
        pl.kernel,
        out_type=jax.ShapeDtypeStruct((E, D), jnp.float32),
        mesh=_mesh(),
        scratch_types=[pltpu.VMEM_SHARED((npad, D), jnp.float32),
                       pltpu.VMEM((nch_s, CHUNK), jnp.int32),
                       pltpu.VMEM((nch_g, CHUNK), jnp.int32),
                       pltpu.VMEM((NBS, CHUNK, D), jnp.float32),
                       pltpu.SemaphoreType.DMA,
                       pltpu.SemaphoreType.DMA,
                       pltpu.SemaphoreType.DMA,
                       pltpu.SemaphoreType.DMA],
    )
    def k(msg_hbm, sidx_hbm, c0_hbm, z_hbm, aggr_hbm,
          S, sidx_v, gidx_v, rbuf, sz, si, sl, sw):
        cid = lax.axis_index("c")
        sid = lax.axis_index("s")
        wid = cid * NS + sid
        dz = pltpu.async_copy(z_hbm, S.at[pl.ds(sid * rpt, rpt)], sz)
        di0 = pltpu.async_copy(sidx_hbm.at[pl.ds(sid * nch_s, nch_s)],
                               sidx_v, si)
        di1 = pltpu.async_copy(c0_hbm.at[pl.ds(wid * nch_g, nch_g)],
                               gidx_v, si)
        ldescs = []
        for c in range(NBS - 1):
            ldescs.append(pltpu.async_copy(
                msg_hbm.at[pl.ds(sid * per_s + c * CHUNK, CHUNK)],
                rbuf.at[c % NBS], sl))
        dz.wait()
        di0.wait()
        di1.wait()
        plsc.subcore_barrier()
        for c in range(nch_s):
            s = c % NBS
            ldescs[c].wait()
            nxt = c + NBS - 1
            if nxt < nch_s:
                ldescs.append(pltpu.async_copy(
                    msg_hbm.at[pl.ds(sid * per_s + nxt * CHUNK, CHUNK)],
                    rbuf.at[nxt % NBS], sl))
            pltpu.sync_copy(rbuf.at[s], S.at[sidx_v.at[c]], add=True)
        plsc.subcore_barrier()
        wdescs = []
        for c in range(nch_g):
            s = c % NBS
            if c >= NBS:
                wdescs[c - NBS].wait()
            pltpu.async_copy(S.at[gidx_v.at[c]], rbuf.at[s], sl).wait()
            wdescs.append(pltpu.async_copy(
                rbuf.at[s],
                aggr_hbm.at[pl.ds(wid * per_g + c * CHUNK, CHUNK)], sw))
        for d in wdescs[max(0, nch_g - NBS):]:
            d.wait()

    return k(msg, sidxr, c0r, zeros_hbm)


def _sc_final(lgX, c1r, zeros_hbm, oblk, npad):
    E, D = lgX.shape
    per = E // NS
    nch = per // CHUNK
    rpt = npad // NS

    @functools.partial(
        pl.kernel,
        out_type=(jax.ShapeDtypeStruct((npad, D), jnp.float32),
                  jax.ShapeDtypeStruct((npad, D), jnp.float32)),
        mesh=_mesh(),
        scratch_types=[pltpu.VMEM_SHARED((npad, D), jnp.float32),
                       pltpu.VMEM((nch, CHUNK), jnp.int32),
                       pltpu.VMEM((NBS, CHUNK, D), jnp.float32),
                       pltpu.SemaphoreType.DMA,
                       pltpu.SemaphoreType.DMA,
                       pltpu.SemaphoreType.DMA,
                       pltpu.SemaphoreType.DMA],
    )
    def k(lg_hbm, c1_hbm, z_hbm, o_hbm, sums_hbm, cnts_hbm,
          S, idx_v, rbuf, sz, si, sl, sw):
        cid = lax.axis_index("c")
        sid = lax.axis_index("s")
        dz = pltpu.async_copy(z_hbm, S.at[pl.ds(sid * rpt, rpt)], sz)
        di = pltpu.async_copy(c1_hbm.at[pl.ds(sid * nch, nch)], idx_v, si)
        dz.wait()
        di.wait()
        plsc.subcore_barrier()

        @pl.when(cid == 0)
        def _():
            ldescs = []
            for c in range(NBS - 1):
                ldescs.append(pltpu.async_copy(
                    lg_hbm.at[pl.ds(sid * per + c * CHUNK, CHUNK)],
                    rbuf.at[c % NBS], sl))
            for c in range(nch):
                ldescs[c].wait()
                nxt = c + NBS - 1
                if nxt < nch:
                    ldescs.append(pltpu.async_copy(
                        lg_hbm.at[pl.ds(sid * per + nxt * CHUNK, CHUNK)],
                        rbuf.at[nxt % NBS], sl))
                pltpu.sync_copy(rbuf.at[c % NBS], S.at[idx_v.at[c]], add=True)

        @pl.when(cid == 1)
        def _():
            pltpu.async_copy(o_hbm, rbuf.at[0], sl).wait()
            for c in range(nch):
                pltpu.sync_copy(rbuf.at[0], S.at[idx_v.at[c]], add=True)

        plsc.subcore_barrier()

        def writeback(dst_hbm):
            wdescs = []
            nwb = rpt // CHUNK
            for c in range(nwb):
                s = c % NBS
                if c >= NBS:
                    wdescs[c - NBS].wait()
                pltpu.async_copy(S.at[pl.ds(sid * rpt + c * CHUNK, CHUNK)],
                                 rbuf.at[s], sl).wait()
                wdescs.append(pltpu.async_copy(
                    rbuf.at[s],
                    dst_hbm.at[pl.ds(sid * rpt + c * CHUNK, CHUNK)], sw))
            for d in wdescs[max(0, nwb - NBS):]:
                d.wait()

        @pl.when(cid == 0)
        def _():
            writeback(sums_hbm)

        @pl.when(cid == 1)
        def _():
            writeback(cnts_hbm)

    return k(lgX, c1r, zeros_hbm, oblk)


_EBLK = 1024


def _tc_prep(xs, xd):
    E, D = xs.shape

    def body(xs_ref, xd_ref, lg_ref, msg_ref):
        s = xs_ref[...]
        d = xd_ref[...]
        lg = (s + d) * 0.5
        lg_ref[...] = lg
        msg_ref[...] = jnp.maximum(lg + d, 0.0)

    return pl.pallas_call(
        body,
        grid=(E // _EBLK,),
        in_specs=[pl.BlockSpec((_EBLK, D), lambda i: (i, 0))] * 2,
        out_specs=[pl.BlockSpec((_EBLK, D), lambda i: (i, 0))] * 2,
        out_shape=[jax.ShapeDtypeStruct((E, D), jnp.float32)] * 2,
    )(xs, xd)


def _tc_update(lgX, aggr, ea, W1, b1r, W2, b2r, want_msg):
    E, D = lgX.shape

    def body(lg_ref, ag_ref, ea_ref, w1_ref, b1_ref, w2_ref, b2_ref, *outs):
        h = lg_ref[...] + ag_ref[...]
        t = jnp.dot(h, w1_ref[...], preferred_element_type=jnp.float32)
        t = jnp.maximum(t + b1_ref[...], 0.0)
        o = jnp.dot(t, w2_ref[...], preferred_element_type=jnp.float32)
        o = o + b2_ref[...]
        outs[0][...] = o
        if want_msg:
            outs[1][...] = jnp.maximum(o + ea_ref[...], 0.0)

    eb = pl.BlockSpec((_EBLK, D), lambda i: (i, 0))
    nout = 2 if want_msg else 1
    return pl.pallas_call(
        body,
        grid=(E // _EBLK,),
        in_specs=[eb, eb, eb,
                  pl.BlockSpec((D, D), lambda i: (0, 0)),
                  pl.BlockSpec((1, D), lambda i: (0, 0)),
                  pl.BlockSpec((D, D), lambda i: (0, 0)),
                  pl.BlockSpec((1, D), lambda i: (0, 0))],
        out_specs=[eb] * nout,
        out_shape=[jax.ShapeDtypeStruct((E, D), jnp.float32)] * nout,
    )(lgX, aggr, ea, W1, b1r, W2, b2r)


def _tc_final(sums, cnts):
    npad, D = sums.shape

    def body(s_ref, c_ref, o_ref):
        c = jnp.maximum(c_ref[...], 1.0)
        o_ref[...] = jnp.maximum(s_ref[...] / c, 0.0)

    nb = pl.BlockSpec((_EBLK, D), lambda i: (i, 0))
    return pl.pallas_call(
        body,
        grid=(npad // _EBLK,),
        in_specs=[nb, nb],
        out_specs=nb,
        out_shape=jax.ShapeDtypeStruct((npad, D), jnp.float32),
    )(sums, cnts)


def kernel(x, edge_index, W1, b1, W2, b2):
    N, D = x.shape
    E = edge_index.shape[1]
    col0 = edge_index[0]
    col1 = edge_index[1]
    sidx = jnp.where(col0 != col1, col1, jnp.int32(N))
    c0r = col0.reshape(E // CHUNK, CHUNK)
    c1r = col1.reshape(E // CHUNK, CHUNK)
    sidxr = sidx.reshape(E // CHUNK, CHUNK)
    npad = -(-(N + 1) // (NS * CHUNK)) * (NS * CHUNK)
    zeros_hbm = jnp.zeros((npad // NS, D), jnp.float32)
    oblk = jnp.ones((CHUNK, D), jnp.float32)
    b1r = b1.reshape(1, D)
    b2r = b2.reshape(1, D)

    xs, xd = _edge_gather(x, c0r, c1r)
    lgX, msg = _tc_prep(xs, xd)
    for r in range(3):
        aggr = _sc_round(msg, sidxr, c0r, zeros_hbm, npad)
        res = _tc_update(lgX, aggr, xd, W1, b1r, W2, b2r, want_msg=(r < 2))
        if r < 2:
            lgX, msg = res
        else:
            (lgX,) = res
    sums, cnts = _sc_final(lgX, c1r, zeros_hbm, oblk, npad)
    out = _tc_final(sums, cnts)
    return out[:N]
